# baseline (device time: 132757 ns/iter reference)
import jax
import jax.numpy as jnp
from jax import lax
from jax.experimental import pallas as pl
from jax.experimental.pallas import tpu as pltpu

N_DEV = 8
N_TOK = 2048
D_IN = 512
D_OUT = 1024
N_EXP = 32
E_LOCAL = N_EXP // N_DEV
CHUNK = N_TOK // N_DEV


def kernel(x, router_W, route_idx, expert_W, shared_W):
    def body(x_ref, rw_ref, idx_ref, ew_ref, sw_ref, out_ref,
             red_ref, rs_buf, rs_send, rs_recv, ag_send, ag_recv):
        pos = lax.axis_index("i")
        left = lax.rem(pos + N_DEV - 1, N_DEV)
        right = lax.rem(pos + 1, N_DEV)

        scores = jnp.dot(x_ref[:, :], rw_ref[:, :],
                         preferred_element_type=jnp.float32)
        s_max = jnp.max(scores, axis=1, keepdims=True)
        ex = jnp.exp(scores - s_max)
        probs = ex / jnp.sum(ex, axis=1, keepdims=True)
        idx = idx_ref[:, :]
        cols = lax.broadcasted_iota(jnp.int32, (N_TOK, N_EXP), 1)
        p_sel = jnp.sum(jnp.where(cols == idx, probs, 0.0), axis=1,
                        keepdims=True)

        x_bf = x_ref[:, :].astype(jnp.bfloat16)
        acc = jnp.zeros((N_TOK, D_OUT), jnp.float32)
        for j in range(E_LOCAL):
            e_id = pos * E_LOCAL + j
            coef = jnp.where(idx == e_id, p_sel, 0.0)
            w = ew_ref[j, :, :].astype(jnp.bfloat16)
            y = jnp.dot(x_bf, w, preferred_element_type=jnp.float32)
            acc = acc + coef * y
        red_ref[:, :] = acc.astype(jnp.bfloat16)

        barrier = pltpu.get_barrier_semaphore()
        for nbr in (left, right):
            pl.semaphore_signal(barrier, inc=1, device_id=(nbr,),
                                device_id_type=pl.DeviceIdType.MESH)
        pl.semaphore_wait(barrier, 2)

        for h in range(N_DEV - 1):
            c_send = lax.rem(pos - h + N_DEV, N_DEV)
            rdma = pltpu.make_async_remote_copy(
                src_ref=red_ref.at[pl.ds(c_send * CHUNK, CHUNK), :],
                dst_ref=rs_buf.at[h],
                send_sem=rs_send.at[h],
                recv_sem=rs_recv.at[h],
                device_id=(right,),
                device_id_type=pl.DeviceIdType.MESH,
            )
            rdma.start()
            rdma.wait()
            c_acc = lax.rem(pos - h - 1 + N_DEV, N_DEV)
            sl = pl.ds(c_acc * CHUNK, CHUNK)
            red_ref[sl, :] = red_ref[sl, :] + rs_buf[h, :, :]

        for s in range(N_DEV - 1):
            c_fwd = lax.rem(pos + 1 - s + N_DEV, N_DEV)
            sl = pl.ds(c_fwd * CHUNK, CHUNK)
            rdma = pltpu.make_async_remote_copy(
                src_ref=red_ref.at[sl, :],
                dst_ref=red_ref.at[sl, :],
                send_sem=ag_send.at[s],
                recv_sem=ag_recv.at[s],
                device_id=(right,),
                device_id_type=pl.DeviceIdType.MESH,
            )
            rdma.start()
            rdma.wait()

        sw_bf = sw_ref[:, :].astype(jnp.bfloat16)
        shared = jnp.dot(x_bf, sw_bf, preferred_element_type=jnp.float32)
        out_ref[:, :] = red_ref[:, :].astype(jnp.float32) + shared

    return pl.pallas_call(
        body,
        out_shape=jax.ShapeDtypeStruct((N_TOK, D_OUT), jnp.float32),
        in_specs=[pl.BlockSpec(memory_space=pltpu.VMEM)] * 5,
        out_specs=pl.BlockSpec(memory_space=pltpu.VMEM),
        scratch_shapes=[
            pltpu.VMEM((N_TOK, D_OUT), jnp.bfloat16),
            pltpu.VMEM((N_DEV - 1, CHUNK, D_OUT), jnp.bfloat16),
            pltpu.SemaphoreType.DMA((N_DEV - 1,)),
            pltpu.SemaphoreType.DMA((N_DEV - 1,)),
            pltpu.SemaphoreType.DMA((N_DEV - 1,)),
            pltpu.SemaphoreType.DMA((N_DEV - 1,)),
        ],
        compiler_params=pltpu.CompilerParams(collective_id=0),
    )(x, router_W, route_idx, expert_W, shared_W)


# device time: 70475 ns/iter; 1.8837x vs baseline; 1.8837x over previous
import jax
import jax.numpy as jnp
from jax import lax
from jax.experimental import pallas as pl
from jax.experimental.pallas import tpu as pltpu

N_DEV = 8
N_TOK = 2048
D_IN = 512
D_OUT = 1024
N_EXP = 32
E_LOCAL = N_EXP // N_DEV
CHUNK = N_TOK // N_DEV

PARTS = ((0, 384), (384, 384), (768, 256))
DIMS = ((1, 3, 4), (3, 4, 1), (4, 1, 3))

RS_BASE = (0, 4, 6)
AG_BASE = (0, 1, 3)


def _xor_span(dims):
    s = {0}
    for d in dims:
        s |= {v ^ d for v in s}
    return sorted(s)


def kernel(x, router_W, route_idx, expert_W, shared_W):
    def body(x_ref, rw_ref, idx_ref, ew_ref, sw_ref, out_ref,
             red_ref, rs_buf, rs_send, rs_recv, ag_send, ag_recv):
        pos = lax.axis_index("i")

        scores = jnp.dot(x_ref[:, :], rw_ref[:, :],
                         preferred_element_type=jnp.float32)
        s_max = jnp.max(scores, axis=1, keepdims=True)
        ex = jnp.exp(scores - s_max)
        probs = ex / jnp.sum(ex, axis=1, keepdims=True)
        idx = idx_ref[:, :]
        cols = lax.broadcasted_iota(jnp.int32, (N_TOK, N_EXP), 1)
        p_sel = jnp.sum(jnp.where(cols == idx, probs, 0.0), axis=1,
                        keepdims=True)

        x_bf = x_ref[:, :].astype(jnp.bfloat16)
        acc = jnp.zeros((N_TOK, D_OUT), jnp.float32)
        for j in range(E_LOCAL):
            e_id = pos * E_LOCAL + j
            coef = jnp.where(idx == e_id, p_sel, 0.0)
            w = ew_ref[j, :, :].astype(jnp.bfloat16)
            y = jnp.dot(x_bf, w, preferred_element_type=jnp.float32)
            acc = acc + coef * y
        red_ref[:, :] = acc.astype(jnp.bfloat16)

        barrier = pltpu.get_barrier_semaphore()
        for g in (1, 3, 4):
            pl.semaphore_signal(barrier, inc=1, device_id=(pos ^ g,),
                                device_id_type=pl.DeviceIdType.MESH)
        pl.semaphore_wait(barrier, 3)

        for r in range(3):
            started = []
            for p, (off, w) in enumerate(PARTS):
                g = DIMS[p][r]
                partner = pos ^ g
                for i, j in enumerate(_xor_span(DIMS[p][r + 1:])):
                    slot = RS_BASE[r] + i
                    sem = p * 7 + slot
                    src_c = partner ^ j
                    d = pltpu.make_async_remote_copy(
                        src_ref=red_ref.at[pl.ds(src_c * CHUNK, CHUNK),
                                           pl.ds(off, w)],
                        dst_ref=rs_buf.at[slot, :, pl.ds(off, w)],
                        send_sem=rs_send.at[sem],
                        recv_sem=rs_recv.at[sem],
                        device_id=(partner,),
                        device_id_type=pl.DeviceIdType.MESH,
                    )
                    d.start()
                    started.append((p, off, w, j, slot, d))
            for _, _, _, _, _, d in started:
                d.wait()
            for p, off, w, j, slot, _ in started:
                rs = pl.ds((pos ^ j) * CHUNK, CHUNK)
                cs = pl.ds(off, w)
                red_ref[rs, cs] = red_ref[rs, cs] + rs_buf[slot, :, cs]

        for k in range(3):
            started = []
            for p, (off, w) in enumerate(PARTS):
                g = DIMS[p][2 - k]
                partner = pos ^ g
                for i, j in enumerate(_xor_span(DIMS[p][3 - k:])):
                    sem = p * 7 + AG_BASE[k] + i
                    sl = (pl.ds((pos ^ j) * CHUNK, CHUNK), pl.ds(off, w))
                    d = pltpu.make_async_remote_copy(
                        src_ref=red_ref.at[sl],
                        dst_ref=red_ref.at[sl],
                        send_sem=ag_send.at[sem],
                        recv_sem=ag_recv.at[sem],
                        device_id=(partner,),
                        device_id_type=pl.DeviceIdType.MESH,
                    )
                    d.start()
                    started.append(d)
            for d in started:
                d.wait()

        sw_bf = sw_ref[:, :].astype(jnp.bfloat16)
        shared = jnp.dot(x_bf, sw_bf, preferred_element_type=jnp.float32)
        out_ref[:, :] = red_ref[:, :].astype(jnp.float32) + shared

    return pl.pallas_call(
        body,
        out_shape=jax.ShapeDtypeStruct((N_TOK, D_OUT), jnp.float32),
        in_specs=[pl.BlockSpec(memory_space=pltpu.VMEM)] * 5,
        out_specs=pl.BlockSpec(memory_space=pltpu.VMEM),
        scratch_shapes=[
            pltpu.VMEM((N_TOK, D_OUT), jnp.bfloat16),
            pltpu.VMEM((7, CHUNK, D_OUT), jnp.bfloat16),
            pltpu.SemaphoreType.DMA((21,)),
            pltpu.SemaphoreType.DMA((21,)),
            pltpu.SemaphoreType.DMA((21,)),
            pltpu.SemaphoreType.DMA((21,)),
        ],
        compiler_params=pltpu.CompilerParams(collective_id=0),
    )(x, router_W, route_idx, expert_W, shared_W)


# device time: 70367 ns/iter; 1.8866x vs baseline; 1.0015x over previous
import jax
import jax.numpy as jnp
from jax import lax
from jax.experimental import pallas as pl
from jax.experimental.pallas import tpu as pltpu

N_DEV = 8
N_TOK = 2048
D_IN = 512
D_OUT = 1024
N_EXP = 32
E_LOCAL = N_EXP // N_DEV
CHUNK = N_TOK // N_DEV

PARTS = ((0, 384), (384, 384), (768, 256))
DIMS = ((1, 3, 4), (3, 4, 1), (4, 1, 3))

RS_BASE = (0, 4, 6)
AG_BASE = (0, 1, 3)


def _xor_span(dims):
    s = {0}
    for d in dims:
        s |= {v ^ d for v in s}
    return sorted(s)


def kernel(x, router_W, route_idx, expert_W, shared_W):
    def body(x_ref, rw_ref, idx_ref, ew_ref, sw_ref, out_ref,
             red_ref, rs_buf, rs_send, rs_recv, ag_send, ag_recv):
        pos = lax.axis_index("i")

        scores = jnp.dot(x_ref[:, :], rw_ref[:, :],
                         preferred_element_type=jnp.float32)
        s_max = jnp.max(scores, axis=1, keepdims=True)
        ex = jnp.exp(scores - s_max)
        probs = ex / jnp.sum(ex, axis=1, keepdims=True)
        idx = idx_ref[:, :]
        cols = lax.broadcasted_iota(jnp.int32, (N_TOK, N_EXP), 1)
        p_sel = jnp.sum(jnp.where(cols == idx, probs, 0.0), axis=1,
                        keepdims=True)

        barrier = pltpu.get_barrier_semaphore()
        for g in (1, 3, 4):
            pl.semaphore_signal(barrier, inc=1, device_id=(pos ^ g,),
                                device_id_type=pl.DeviceIdType.MESH)
        pl.semaphore_wait(barrier, 3)

        def start_round(p, off, w, r, bases, sends, recvs):
            g = DIMS[p][r]
            partner = pos ^ g
            started = []
            for i, j in enumerate(_xor_span(DIMS[p][r + 1:])):
                slot = bases[r] + i
                sem = p * 7 + slot
                src_c = partner ^ j
                d = pltpu.make_async_remote_copy(
                    src_ref=red_ref.at[pl.ds(src_c * CHUNK, CHUNK),
                                       pl.ds(off, w)],
                    dst_ref=rs_buf.at[slot, :, pl.ds(off, w)],
                    send_sem=sends.at[sem],
                    recv_sem=recvs.at[sem],
                    device_id=(partner,),
                    device_id_type=pl.DeviceIdType.MESH,
                )
                d.start()
                started.append((p, off, w, j, slot, d))
            return started

        x_bf = x_ref[:, :].astype(jnp.bfloat16)
        coefs = [jnp.where(idx == pos * E_LOCAL + j, p_sel, 0.0)
                 for j in range(E_LOCAL)]
        r0_started = []
        for p, (off, w) in enumerate(PARTS):
            accp = jnp.zeros((N_TOK, w), jnp.float32)
            for j in range(E_LOCAL):
                wj = ew_ref[j, :, off:off + w].astype(jnp.bfloat16)
                accp = accp + coefs[j] * jnp.dot(
                    x_bf, wj, preferred_element_type=jnp.float32)
            red_ref[:, off:off + w] = accp.astype(jnp.bfloat16)
            r0_started.extend(start_round(p, off, w, 0, RS_BASE,
                                          rs_send, rs_recv))

        sw_bf = sw_ref[:, :].astype(jnp.bfloat16)
        out_ref[:, :] = jnp.dot(x_bf, sw_bf,
                                preferred_element_type=jnp.float32)

        for r in range(3):
            if r == 0:
                started = r0_started
            else:
                started = []
                for p, (off, w) in enumerate(PARTS):
                    started.extend(start_round(p, off, w, r, RS_BASE,
                                               rs_send, rs_recv))
            for _, _, _, _, _, d in started:
                d.wait()
            for p, off, w, j, slot, _ in started:
                rs = pl.ds((pos ^ j) * CHUNK, CHUNK)
                cs = pl.ds(off, w)
                red_ref[rs, cs] = red_ref[rs, cs] + rs_buf[slot, :, cs]

        for k in range(3):
            started = []
            for p, (off, w) in enumerate(PARTS):
                g = DIMS[p][2 - k]
                partner = pos ^ g
                for i, j in enumerate(_xor_span(DIMS[p][3 - k:])):
                    sem = p * 7 + AG_BASE[k] + i
                    sl = (pl.ds((pos ^ j) * CHUNK, CHUNK), pl.ds(off, w))
                    d = pltpu.make_async_remote_copy(
                        src_ref=red_ref.at[sl],
                        dst_ref=red_ref.at[sl],
                        send_sem=ag_send.at[sem],
                        recv_sem=ag_recv.at[sem],
                        device_id=(partner,),
                        device_id_type=pl.DeviceIdType.MESH,
                    )
                    d.start()
                    started.append(d)
            for d in started:
                d.wait()

        out_ref[:, :] = out_ref[:, :] + red_ref[:, :].astype(jnp.float32)

    return pl.pallas_call(
        body,
        out_shape=jax.ShapeDtypeStruct((N_TOK, D_OUT), jnp.float32),
        in_specs=[pl.BlockSpec(memory_space=pltpu.VMEM)] * 5,
        out_specs=pl.BlockSpec(memory_space=pltpu.VMEM),
        scratch_shapes=[
            pltpu.VMEM((N_TOK, D_OUT), jnp.bfloat16),
            pltpu.VMEM((7, CHUNK, D_OUT), jnp.bfloat16),
            pltpu.SemaphoreType.DMA((21,)),
            pltpu.SemaphoreType.DMA((21,)),
            pltpu.SemaphoreType.DMA((21,)),
            pltpu.SemaphoreType.DMA((21,)),
        ],
        compiler_params=pltpu.CompilerParams(collective_id=0),
    )(x, router_W, route_idx, expert_W, shared_W)


# device time: 58378 ns/iter; 2.2741x vs baseline; 1.2054x over previous
import os

import jax
import jax.numpy as jnp
from jax import lax
from jax.experimental import pallas as pl
from jax.experimental.pallas import tpu as pltpu

N_DEV = 8
N_TOK = 2048
D_IN = 512
D_OUT = 1024
N_EXP = 32
E_LOCAL = N_EXP // N_DEV
CHUNK = N_TOK // N_DEV

PARTS = ((0, 384), (384, 384), (768, 256))
DIMS = ((1, 3, 4), (3, 4, 1), (4, 1, 3))

RS_BASE = (0, 4, 6)
AG_BASE = (0, 1, 3)


_KMODE = os.environ.get("KMODE", "full")


def _xor_span(dims):
    s = {0}
    for d in dims:
        s |= {v ^ d for v in s}
    return sorted(s)


def kernel(x, router_W, route_idx, expert_W, shared_W):
    def body(x_ref, rw_ref, idx_ref, ew_ref, sw_ref, out_ref,
             red_ref, rs_buf, rs_send, rs_recv, ag_send, ag_recv):
        pos = lax.axis_index("i")

        scores = jnp.dot(x_ref[:, :], rw_ref[:, :],
                         preferred_element_type=jnp.float32)
        s_max = jnp.max(scores, axis=1, keepdims=True)
        ex = jnp.exp(scores - s_max)
        probs = ex / jnp.sum(ex, axis=1, keepdims=True)
        idx = idx_ref[:, :]
        cols = lax.broadcasted_iota(jnp.int32, (N_TOK, N_EXP), 1)
        p_sel = jnp.sum(jnp.where(cols == idx, probs, 0.0), axis=1,
                        keepdims=True)

        if _KMODE != "compute_only":
            barrier = pltpu.get_barrier_semaphore()
            for g in (1, 3, 4):
                pl.semaphore_signal(barrier, inc=1, device_id=(pos ^ g,),
                                    device_id_type=pl.DeviceIdType.MESH)
            pl.semaphore_wait(barrier, 3)

        def start_round(p, off, w, r, bases, sends, recvs):
            g = DIMS[p][r]
            partner = pos ^ g
            started = []
            for i, j in enumerate(_xor_span(DIMS[p][r + 1:])):
                slot = bases[r] + i
                sem = p * 7 + slot
                src_c = partner ^ j
                d = pltpu.make_async_remote_copy(
                    src_ref=red_ref.at[pl.ds(src_c * CHUNK, CHUNK),
                                       pl.ds(off, w)],
                    dst_ref=rs_buf.at[slot, :, pl.ds(off, w)],
                    send_sem=sends.at[sem],
                    recv_sem=recvs.at[sem],
                    device_id=(partner,),
                    device_id_type=pl.DeviceIdType.MESH,
                )
                d.start()
                started.append((p, off, w, j, slot, d))
            return started

        x_bf = x_ref[:, :].astype(jnp.bfloat16)
        coefs = [jnp.where(idx == pos * E_LOCAL + j, p_sel, 0.0)
                 for j in range(E_LOCAL)]
        r0_started = []
        for p, (off, w) in enumerate(PARTS):
            if _KMODE == "comm_only":
                red_ref[:, off:off + w] = jnp.zeros((N_TOK, w), jnp.bfloat16)
            else:
                accp = jnp.zeros((N_TOK, w), jnp.float32)
                for j in range(E_LOCAL):
                    wj = ew_ref[j, :, off:off + w].astype(jnp.bfloat16)
                    accp = accp + coefs[j] * jnp.dot(
                        x_bf, wj, preferred_element_type=jnp.float32)
                red_ref[:, off:off + w] = accp.astype(jnp.bfloat16)
            if _KMODE != "compute_only":
                r0_started.extend(start_round(p, off, w, 0, RS_BASE,
                                              rs_send, rs_recv))

        if _KMODE == "comm_only":
            out_ref[:, :] = jnp.zeros((N_TOK, D_OUT), jnp.float32)
        else:
            sw_bf = sw_ref[:, :].astype(jnp.bfloat16)
            out_ref[:, :] = jnp.dot(x_bf, sw_bf,
                                    preferred_element_type=jnp.float32)

        for r in range(3 if _KMODE != "compute_only" else 0):
            if r == 0:
                started = r0_started
            else:
                started = []
                for p, (off, w) in enumerate(PARTS):
                    started.extend(start_round(p, off, w, r, RS_BASE,
                                               rs_send, rs_recv))
            for _, _, _, _, _, d in started:
                d.wait()
            for p, off, w, j, slot, _ in started:
                rs = pl.ds((pos ^ j) * CHUNK, CHUNK)
                cs = pl.ds(off, w)
                red_ref[rs, cs] = red_ref[rs, cs] + rs_buf[slot, :, cs]

        for k in range(3 if _KMODE != "compute_only" else 0):
            started = []
            for p, (off, w) in enumerate(PARTS):
                g = DIMS[p][2 - k]
                partner = pos ^ g
                for i, j in enumerate(_xor_span(DIMS[p][3 - k:])):
                    sem = p * 7 + AG_BASE[k] + i
                    sl = (pl.ds((pos ^ j) * CHUNK, CHUNK), pl.ds(off, w))
                    d = pltpu.make_async_remote_copy(
                        src_ref=red_ref.at[sl],
                        dst_ref=red_ref.at[sl],
                        send_sem=ag_send.at[sem],
                        recv_sem=ag_recv.at[sem],
                        device_id=(partner,),
                        device_id_type=pl.DeviceIdType.MESH,
                    )
                    d.start()
                    started.append(d)
            for d in started:
                d.wait()

        out_ref[:, :] = out_ref[:, :] + red_ref[:, :].astype(jnp.float32)

    return pl.pallas_call(
        body,
        out_shape=jax.ShapeDtypeStruct((N_TOK, D_OUT), jnp.float32),
        in_specs=[pl.BlockSpec(memory_space=pltpu.VMEM)] * 5,
        out_specs=pl.BlockSpec(memory_space=pltpu.VMEM),
        scratch_shapes=[
            pltpu.VMEM((N_TOK, D_OUT), jnp.bfloat16),
            pltpu.VMEM((7, CHUNK, D_OUT), jnp.bfloat16),
            pltpu.SemaphoreType.DMA((21,)),
            pltpu.SemaphoreType.DMA((21,)),
            pltpu.SemaphoreType.DMA((21,)),
            pltpu.SemaphoreType.DMA((21,)),
        ],
        compiler_params=pltpu.CompilerParams(collective_id=0),
    )(x, router_W, route_idx, expert_W, shared_W)
